# Initial kernel scaffold; baseline (speedup 1.0000x reference)
import jax
import jax.numpy as jnp
from jax.experimental import pallas as pl


def kernel(weights, p1_idx, p2_idx, s1, total_matches):
    raise NotImplementedError("write your pallas kernel here")


# speedup vs baseline: 110.9557x; 110.9557x over previous
import functools

import jax
import jax.numpy as jnp
from jax import lax
from jax.experimental import pallas as pl
from jax.experimental.pallas import tpu as pltpu
from jax.experimental.pallas import tpu_sc as plsc
from jax.scipy.special import gammaln

N_PLAYERS = 100000
N_MATCH = 1600000

NC = 2
NS = 16
L = 16
NW = NC * NS
PER_W = N_MATCH // NW
CB = 2000
NCHUNK = PER_W // CB
VPC = CB // L

LN2 = 0.6931471805599453
SQRTHF2 = 1.4142135623730951

_LOG_P = (
    7.0376836292e-2,
    -1.1514610310e-1,
    1.1676998740e-1,
    -1.2420140846e-1,
    1.4249322787e-1,
    -1.6668057665e-1,
    2.0000714765e-1,
    -2.4999993993e-1,
    3.3333331174e-1,
)


def _log16(y):
    bits = lax.bitcast_convert_type(y, jnp.int32)
    e = jnp.right_shift(bits, 23) - 127
    mbits = jnp.bitwise_or(jnp.bitwise_and(bits, 0x007FFFFF), 0x3F800000)
    m = lax.bitcast_convert_type(mbits, jnp.float32)
    big = m > SQRTHF2
    m = jnp.where(big, m * 0.5, m)
    e = e + big.astype(jnp.int32)
    t = m - 1.0
    p = jnp.full((L,), _LOG_P[0], jnp.float32)
    for c in _LOG_P[1:]:
        p = p * t + c
    z = t * t
    logm = t * (z * p) - 0.5 * z + t
    return logm + e.astype(jnp.float32) * LN2


def _tec_body(w_hbm, p1_hbm, p2_hbm, s1_hbm, tm_hbm, lgf_hbm,
              elem_hbm, csum_hbm,
              table_v, lgf_v, i1_v, i2_v, s_v, t_v, o_v, red_v, sidx_v,
              shared_sp, sem):
    c = lax.axis_index("c")
    s = lax.axis_index("s")
    wid = c * NS + s
    base = wid * PER_W

    htab = pltpu.async_copy(w_hbm, table_v, sem)
    hlgf = pltpu.async_copy(lgf_hbm, lgf_v, sem)
    htab.wait()
    hlgf.wait()

    def chunk_body(ci, acc):
        off = base + ci * CB
        h1 = pltpu.async_copy(p1_hbm.at[pl.ds(off, CB)], i1_v, sem)
        h2 = pltpu.async_copy(p2_hbm.at[pl.ds(off, CB)], i2_v, sem)
        h3 = pltpu.async_copy(s1_hbm.at[pl.ds(off, CB)], s_v, sem)
        h4 = pltpu.async_copy(tm_hbm.at[pl.ds(off, CB)], t_v, sem)
        h1.wait()
        h2.wait()
        h3.wait()
        h4.wait()

        def vec_body(j, acc):
            o = j * L
            i1 = i1_v[pl.ds(o, L)]
            i2 = i2_v[pl.ds(o, L)]
            w1 = plsc.load_gather(table_v, [i1])
            w2 = plsc.load_gather(table_v, [i2])
            sf = s_v[pl.ds(o, L)]
            nf = t_v[pl.ds(o, L)]
            si = sf.astype(jnp.int32)
            ni = nf.astype(jnp.int32)
            lA = plsc.load_gather(lgf_v, [ni])
            lB = plsc.load_gather(lgf_v, [si])
            lC = plsc.load_gather(lgf_v, [ni - si])
            d = w1 - w2
            ed = jnp.exp(d)
            Lv = _log16(ed + 1.0)
            elem = (lA - lB - lC) + sf * d - nf * Lv
            o_v[pl.ds(o, L)] = elem
            return acc + elem

        acc = lax.fori_loop(0, VPC, vec_body, acc)
        pltpu.sync_copy(o_v, elem_hbm.at[pl.ds(off, CB)])
        return acc

    acc = lax.fori_loop(0, NCHUNK, chunk_body, jnp.zeros((L,), jnp.float32))

    @pl.when(s == 0)
    def _init():
        red_v[...] = jnp.zeros((L,), jnp.float32)
        pltpu.sync_copy(red_v, shared_sp)

    plsc.subcore_barrier()
    sidx_v[...] = lax.iota(jnp.int32, L)
    red_v[...] = acc
    pltpu.sync_copy(red_v, shared_sp.at[sidx_v], add=True)
    plsc.subcore_barrier()

    @pl.when(s == 0)
    def _emit():
        pltpu.sync_copy(shared_sp, red_v)
        tot = jnp.sum(red_v[...])
        red_v[...] = jnp.full((L,), -tot, jnp.float32)
        pltpu.sync_copy(red_v, csum_hbm.at[c])


@functools.cache
def _get_loss_kernel():
  return pl.kernel(
    _tec_body,
    out_type=(
        jax.ShapeDtypeStruct((N_MATCH,), jnp.float32),
        jax.ShapeDtypeStruct((NC, L), jnp.float32),
    ),
    mesh=plsc.VectorSubcoreMesh(
        core_axis_name="c", subcore_axis_name="s",
        num_cores=NC, num_subcores=NS),
    compiler_params=pltpu.CompilerParams(needs_layout_passes=False),
    scratch_types=[
        pltpu.VMEM((N_PLAYERS,), jnp.float32),
        pltpu.VMEM((64,), jnp.float32),
        pltpu.VMEM((CB,), jnp.int32),
        pltpu.VMEM((CB,), jnp.int32),
        pltpu.VMEM((CB,), jnp.float32),
        pltpu.VMEM((CB,), jnp.float32),
        pltpu.VMEM((CB,), jnp.float32),
        pltpu.VMEM((L,), jnp.float32),
        pltpu.VMEM((L,), jnp.int32),
        pltpu.VMEM_SHARED((L,), jnp.float32),
        pltpu.SemaphoreType.DMA,
    ],
  )


@jax.jit
def kernel(weights, p1_idx, p2_idx, s1, total_matches):
    lgf = gammaln(jnp.arange(1, 65, dtype=jnp.float32))
    elem, csum = _get_loss_kernel()(weights, p1_idx, p2_idx, s1, total_matches, lgf)
    loss_val = csum[0, 0] + csum[1, 0]
    return (loss_val, elem)


# speedup vs baseline: 123.8498x; 1.1162x over previous
import functools

import jax
import jax.numpy as jnp
from jax import lax
from jax.experimental import pallas as pl
from jax.experimental.pallas import tpu as pltpu
from jax.experimental.pallas import tpu_sc as plsc
from jax.scipy.special import gammaln

N_PLAYERS = 100000
N_MATCH = 1600000

NC = 2
NS = 16
L = 16
NW = NC * NS
PER_W = N_MATCH // NW
CB = 2000
NCHUNK = PER_W // CB
VPC = CB // L
UNROLL = 5
NBUF = 2

LN2 = 0.6931471805599453
SQRT2 = 1.4142135623730951

_LOG_P = (
    7.0376836292e-2,
    -1.1514610310e-1,
    1.1676998740e-1,
    -1.2420140846e-1,
    1.4249322787e-1,
    -1.6668057665e-1,
    2.0000714765e-1,
    -2.4999993993e-1,
    3.3333331174e-1,
)


def _log16(y):
    bits = lax.bitcast_convert_type(y, jnp.int32)
    e = jnp.right_shift(bits, 23) - 127
    mbits = jnp.bitwise_or(jnp.bitwise_and(bits, 0x007FFFFF), 0x3F800000)
    m = lax.bitcast_convert_type(mbits, jnp.float32)
    big = m > SQRT2
    m = jnp.where(big, m * 0.5, m)
    e = e + big.astype(jnp.int32)
    t = m - 1.0
    p = jnp.full((L,), _LOG_P[0], jnp.float32)
    for c in _LOG_P[1:]:
        p = p * t + c
    z = t * t
    logm = t * (z * p) - 0.5 * z + t
    return logm + e.astype(jnp.float32) * LN2


def _tec_body(w_hbm, p1_hbm, p2_hbm, s1_hbm, tm_hbm, lgc_hbm,
              elem_hbm, csum_hbm,
              table_v, lgc_v, i1a, i1b, i2a, i2b, sa, sb, ta, tb, oa, ob,
              red_v, sidx_v,
              shared_sp, sem_tab, sin0, sin1, sout0, sout1):
    c = lax.axis_index("c")
    s = lax.axis_index("s")
    wid = c * NS + s
    base = wid * PER_W
    sin = (sin0, sin1)
    sout = (sout0, sout1)
    i1_v = (i1a, i1b)
    i2_v = (i2a, i2b)
    s_v = (sa, sb)
    t_v = (ta, tb)
    o_v = (oa, ob)

    htab = pltpu.async_copy(w_hbm, table_v, sem_tab)
    hlgc = pltpu.async_copy(lgc_hbm, lgc_v, sem_tab)

    def start_in(cur, b):
        off = base + cur * CB
        pltpu.async_copy(p1_hbm.at[pl.ds(off, CB)], i1_v[b], sin[b])
        pltpu.async_copy(p2_hbm.at[pl.ds(off, CB)], i2_v[b], sin[b])
        pltpu.async_copy(s1_hbm.at[pl.ds(off, CB)], s_v[b], sin[b])
        pltpu.async_copy(tm_hbm.at[pl.ds(off, CB)], t_v[b], sin[b])

    def wait_in(b):
        pltpu.make_async_copy(p1_hbm.at[pl.ds(0, CB)], i1_v[b], sin[b]).wait()
        pltpu.make_async_copy(p2_hbm.at[pl.ds(0, CB)], i2_v[b], sin[b]).wait()
        pltpu.make_async_copy(s1_hbm.at[pl.ds(0, CB)], s_v[b], sin[b]).wait()
        pltpu.make_async_copy(tm_hbm.at[pl.ds(0, CB)], t_v[b], sin[b]).wait()

    def wait_out(b):
        pltpu.make_async_copy(
            o_v[b], elem_hbm.at[pl.ds(0, CB)], sout[b]).wait()

    def compute(b, acc):
        def vec_body(j, acc):
            for u in range(UNROLL):
                o = j * (L * UNROLL) + u * L
                i1 = i1_v[b][pl.ds(o, L)]
                i2 = i2_v[b][pl.ds(o, L)]
                w1 = plsc.load_gather(table_v, [i1])
                w2 = plsc.load_gather(table_v, [i2])
                sf = s_v[b][pl.ds(o, L)]
                nf = t_v[b][pl.ds(o, L)]
                si = sf.astype(jnp.int32)
                ni = nf.astype(jnp.int32)
                lgc = plsc.load_gather(
                    lgc_v, [jnp.bitwise_or(lax.shift_left(ni, 6), si)])
                d = w1 - w2
                ed = jnp.exp(d)
                Lv = _log16(ed + 1.0)
                elem = lgc + sf * d - nf * Lv
                o_v[b][pl.ds(o, L)] = elem
                acc = acc + elem
            return acc
        return lax.fori_loop(0, VPC // UNROLL, vec_body, acc)

    def do_chunk(cur, b, acc, first_use_at):
        off = base + cur * CB
        wait_in(b)

        @pl.when(cur > first_use_at)
        def _():
            wait_out(b)

        acc = compute(b, acc)
        pltpu.async_copy(o_v[b], elem_hbm.at[pl.ds(off, CB)], sout[b])

        @pl.when(cur + NBUF < NCHUNK)
        def _():
            start_in(cur + NBUF, b)

        return acc

    start_in(0, 0)
    start_in(1, 1)
    htab.wait()
    hlgc.wait()

    def pair_body(r, acc):
        acc = do_chunk(2 * r, 0, acc, 0)
        acc = do_chunk(2 * r + 1, 1, acc, 1)
        return acc

    acc = lax.fori_loop(0, (NCHUNK - 1) // NBUF,
                        pair_body, jnp.zeros((L,), jnp.float32))
    acc = do_chunk(NCHUNK - 1, 0, acc, 0)
    wait_out(0)
    wait_out(1)

    @pl.when(s == 0)
    def _init():
        red_v[...] = jnp.zeros((L,), jnp.float32)
        pltpu.sync_copy(red_v, shared_sp)

    plsc.subcore_barrier()
    sidx_v[...] = lax.iota(jnp.int32, L)
    red_v[...] = acc
    pltpu.sync_copy(red_v, shared_sp.at[sidx_v], add=True)
    plsc.subcore_barrier()

    @pl.when(s == 0)
    def _emit():
        pltpu.sync_copy(shared_sp, red_v)
        tot = jnp.sum(red_v[...])
        red_v[...] = jnp.full((L,), -tot, jnp.float32)
        pltpu.sync_copy(red_v, csum_hbm.at[c])


@functools.cache
def _get_loss_kernel():
  return pl.kernel(
    _tec_body,
    out_type=(
        jax.ShapeDtypeStruct((N_MATCH,), jnp.float32),
        jax.ShapeDtypeStruct((NC, L), jnp.float32),
    ),
    mesh=plsc.VectorSubcoreMesh(
        core_axis_name="c", subcore_axis_name="s",
        num_cores=NC, num_subcores=NS),
    compiler_params=pltpu.CompilerParams(needs_layout_passes=False),
    scratch_types=[
        pltpu.VMEM((N_PLAYERS,), jnp.float32),
        pltpu.VMEM((64 * 64,), jnp.float32),
        pltpu.VMEM((CB,), jnp.int32),
        pltpu.VMEM((CB,), jnp.int32),
        pltpu.VMEM((CB,), jnp.int32),
        pltpu.VMEM((CB,), jnp.int32),
        pltpu.VMEM((CB,), jnp.float32),
        pltpu.VMEM((CB,), jnp.float32),
        pltpu.VMEM((CB,), jnp.float32),
        pltpu.VMEM((CB,), jnp.float32),
        pltpu.VMEM((CB,), jnp.float32),
        pltpu.VMEM((CB,), jnp.float32),
        pltpu.VMEM((L,), jnp.float32),
        pltpu.VMEM((L,), jnp.int32),
        pltpu.VMEM_SHARED((L,), jnp.float32),
        pltpu.SemaphoreType.DMA,
        pltpu.SemaphoreType.DMA,
        pltpu.SemaphoreType.DMA,
        pltpu.SemaphoreType.DMA,
        pltpu.SemaphoreType.DMA,
    ],
  )


@jax.jit
def kernel(weights, p1_idx, p2_idx, s1, total_matches):
    k = jnp.arange(64, dtype=jnp.float32)
    n2 = k[:, None]
    s2 = k[None, :]
    lgc = ((gammaln(n2 + 1.0) - gammaln(s2 + 1.0))
           - gammaln(n2 - s2 + 1.0)).reshape(-1)
    elem, csum = _get_loss_kernel()(
        weights, p1_idx, p2_idx, s1, total_matches, lgc)
    loss_val = csum[0, 0] + csum[1, 0]
    return (loss_val, elem)


# speedup vs baseline: 317.0774x; 2.5602x over previous
import functools

import jax
import jax.numpy as jnp
from jax import lax
from jax.experimental import pallas as pl
from jax.experimental.pallas import tpu as pltpu
from jax.experimental.pallas import tpu_sc as plsc
from jax.scipy.special import gammaln

N_PLAYERS = 100000
N_MATCH = 1600000

NC = 2
NS = 16
L = 16
NW = NC * NS
PER_W = N_MATCH // NW
CB = 2000
NCHUNK = PER_W // CB
VPC = CB // L
UNROLL = 5
NBUF = 2

LN2 = 0.6931471805599453
SQRT2 = 1.4142135623730951

_LOG_P = (
    7.0376836292e-2,
    -1.1514610310e-1,
    1.1676998740e-1,
    -1.2420140846e-1,
    1.4249322787e-1,
    -1.6668057665e-1,
    2.0000714765e-1,
    -2.4999993993e-1,
    3.3333331174e-1,
)


def _log16(y):
    bits = lax.bitcast_convert_type(y, jnp.int32)
    e = jnp.right_shift(bits, 23) - 127
    mbits = jnp.bitwise_or(jnp.bitwise_and(bits, 0x007FFFFF), 0x3F800000)
    m = lax.bitcast_convert_type(mbits, jnp.float32)
    big = m > SQRT2
    m = jnp.where(big, m * 0.5, m)
    e = e + big.astype(jnp.int32)
    t = m - 1.0
    p = jnp.full((L,), _LOG_P[0], jnp.float32)
    for c in _LOG_P[1:]:
        p = p * t + c
    z = t * t
    logm = t * (z * p) - 0.5 * z + t
    return logm + e.astype(jnp.float32) * LN2


def _tec_body(w_hbm, p1_hbm, p2_hbm, s1_hbm, tm_hbm, lgc_hbm,
              elem_hbm, csum_hbm,
              table_v, lgc_v, i1a, i1b, i2a, i2b, sa, sb, ta, tb, oa, ob,
              red_v, sidx_v,
              shared_sp, sem_tab, sin0, sin1, sout0, sout1):
    c = lax.axis_index("c")
    s = lax.axis_index("s")
    wid = c * NS + s
    base = wid * PER_W
    sin = (sin0, sin1)
    sout = (sout0, sout1)
    i1_v = (i1a, i1b)
    i2_v = (i2a, i2b)
    s_v = (sa, sb)
    t_v = (ta, tb)
    o_v = (oa, ob)

    htab = pltpu.async_copy(w_hbm, table_v, sem_tab)
    hlgc = pltpu.async_copy(lgc_hbm, lgc_v, sem_tab)

    def start_in(cur, b):
        off = base + cur * CB
        pltpu.async_copy(p1_hbm.at[pl.ds(off, CB)], i1_v[b], sin[b])
        pltpu.async_copy(p2_hbm.at[pl.ds(off, CB)], i2_v[b], sin[b])
        pltpu.async_copy(s1_hbm.at[pl.ds(off, CB)], s_v[b], sin[b])
        pltpu.async_copy(tm_hbm.at[pl.ds(off, CB)], t_v[b], sin[b])

    def wait_in(b):
        pltpu.make_async_copy(p1_hbm.at[pl.ds(0, CB)], i1_v[b], sin[b]).wait()
        pltpu.make_async_copy(p2_hbm.at[pl.ds(0, CB)], i2_v[b], sin[b]).wait()
        pltpu.make_async_copy(s1_hbm.at[pl.ds(0, CB)], s_v[b], sin[b]).wait()
        pltpu.make_async_copy(tm_hbm.at[pl.ds(0, CB)], t_v[b], sin[b]).wait()

    def wait_out(b):
        pltpu.make_async_copy(
            o_v[b], elem_hbm.at[pl.ds(0, CB)], sout[b]).wait()

    def compute(b, acc):
        def vec_body(o, acc):
            i1 = i1_v[b][pl.ds(o, L)]
            i2 = i2_v[b][pl.ds(o, L)]
            w1 = plsc.load_gather(table_v, [i1])
            w2 = plsc.load_gather(table_v, [i2])
            sf = s_v[b][pl.ds(o, L)]
            nf = t_v[b][pl.ds(o, L)]
            si = sf.astype(jnp.int32)
            ni = nf.astype(jnp.int32)
            lgc = plsc.load_gather(
                lgc_v, [jnp.bitwise_or(lax.shift_left(ni, 6), si)])
            d = w1 - w2
            ed = jnp.exp(d)
            Lv = _log16(ed + 1.0)
            elem = lgc + sf * d - nf * Lv
            o_v[b][pl.ds(o, L)] = elem
            return acc + elem
        return plsc.parallel_loop(0, CB, L, unroll=UNROLL, carry=acc)(vec_body)

    def do_chunk(cur, b, acc, first_use_at):
        off = base + cur * CB
        wait_in(b)

        @pl.when(cur > first_use_at)
        def _():
            wait_out(b)

        acc = compute(b, acc)
        pltpu.async_copy(o_v[b], elem_hbm.at[pl.ds(off, CB)], sout[b])

        @pl.when(cur + NBUF < NCHUNK)
        def _():
            start_in(cur + NBUF, b)

        return acc

    start_in(0, 0)
    start_in(1, 1)
    htab.wait()
    hlgc.wait()

    def pair_body(r, acc):
        acc = do_chunk(2 * r, 0, acc, 0)
        acc = do_chunk(2 * r + 1, 1, acc, 1)
        return acc

    acc = lax.fori_loop(0, (NCHUNK - 1) // NBUF,
                        pair_body, jnp.zeros((L,), jnp.float32))
    acc = do_chunk(NCHUNK - 1, 0, acc, 0)
    wait_out(0)
    wait_out(1)

    @pl.when(s == 0)
    def _init():
        red_v[...] = jnp.zeros((L,), jnp.float32)
        pltpu.sync_copy(red_v, shared_sp)

    plsc.subcore_barrier()
    sidx_v[...] = lax.iota(jnp.int32, L)
    red_v[...] = acc
    pltpu.sync_copy(red_v, shared_sp.at[sidx_v], add=True)
    plsc.subcore_barrier()

    @pl.when(s == 0)
    def _emit():
        pltpu.sync_copy(shared_sp, red_v)
        tot = jnp.sum(red_v[...])
        red_v[...] = jnp.full((L,), -tot, jnp.float32)
        pltpu.sync_copy(red_v, csum_hbm.at[c])


@functools.cache
def _get_loss_kernel():
  return pl.kernel(
    _tec_body,
    out_type=(
        jax.ShapeDtypeStruct((N_MATCH,), jnp.float32),
        jax.ShapeDtypeStruct((NC, L), jnp.float32),
    ),
    mesh=plsc.VectorSubcoreMesh(
        core_axis_name="c", subcore_axis_name="s",
        num_cores=NC, num_subcores=NS),
    compiler_params=pltpu.CompilerParams(needs_layout_passes=False),
    scratch_types=[
        pltpu.VMEM((N_PLAYERS,), jnp.float32),
        pltpu.VMEM((64 * 64,), jnp.float32),
        pltpu.VMEM((CB,), jnp.int32),
        pltpu.VMEM((CB,), jnp.int32),
        pltpu.VMEM((CB,), jnp.int32),
        pltpu.VMEM((CB,), jnp.int32),
        pltpu.VMEM((CB,), jnp.float32),
        pltpu.VMEM((CB,), jnp.float32),
        pltpu.VMEM((CB,), jnp.float32),
        pltpu.VMEM((CB,), jnp.float32),
        pltpu.VMEM((CB,), jnp.float32),
        pltpu.VMEM((CB,), jnp.float32),
        pltpu.VMEM((L,), jnp.float32),
        pltpu.VMEM((L,), jnp.int32),
        pltpu.VMEM_SHARED((L,), jnp.float32),
        pltpu.SemaphoreType.DMA,
        pltpu.SemaphoreType.DMA,
        pltpu.SemaphoreType.DMA,
        pltpu.SemaphoreType.DMA,
        pltpu.SemaphoreType.DMA,
    ],
  )


@jax.jit
def kernel(weights, p1_idx, p2_idx, s1, total_matches):
    k = jnp.arange(64, dtype=jnp.float32)
    n2 = k[:, None]
    s2 = k[None, :]
    lgc = ((gammaln(n2 + 1.0) - gammaln(s2 + 1.0))
           - gammaln(n2 - s2 + 1.0)).reshape(-1)
    elem, csum = _get_loss_kernel()(
        weights, p1_idx, p2_idx, s1, total_matches, lgc)
    loss_val = csum[0, 0] + csum[1, 0]
    return (loss_val, elem)
